# Initial kernel scaffold; baseline (speedup 1.0000x reference)
import jax
import jax.numpy as jnp
from jax.experimental import pallas as pl


def kernel(feature, centers):
    raise NotImplementedError("write your pallas kernel here")


# speedup vs baseline: 605.2379x; 605.2379x over previous
import jax
import jax.numpy as jnp
from jax.experimental import pallas as pl
from jax.experimental.pallas import tpu as pltpu

_MARGIN = 1.0
_MAX_ITER = 15
_NEAREST = 3
_NUM = 2
_INF = float("inf")


def _ipfl_body(f_ref, c_ref, out_ref):
    f = f_ref[:]
    c = c_ref[:]
    B, K = f.shape
    C = c.shape[0]

    hi = jax.lax.Precision.HIGHEST
    fn = jnp.sum(f * f, axis=1, keepdims=True)
    ones_row = jnp.ones((1, K), jnp.float32)
    cn_row = jax.lax.dot_general(
        ones_row, c * c, (((1,), (1,)), ((), ())),
        precision=hi, preferred_element_type=jnp.float32)
    fc = jax.lax.dot_general(
        f, c, (((1,), (1,)), ((), ())),
        precision=hi, preferred_element_type=jnp.float32)
    D2 = jnp.maximum(fn + cn_row - 2.0 * fc, 0.0)
    D = jnp.sqrt(D2)

    cc = jax.lax.dot_general(
        c, c, (((1,), (1,)), ((), ())),
        precision=hi, preferred_element_type=jnp.float32)
    cn_col = jax.lax.dot_general(
        c * c, ones_row, (((1,), (1,)), ((), ())),
        precision=hi, preferred_element_type=jnp.float32)
    S2 = jnp.maximum(cn_col + cn_row - 2.0 * cc, 0.0)
    rowc = jax.lax.broadcasted_iota(jnp.int32, (C, C), 0)
    colc = jax.lax.broadcasted_iota(jnp.int32, (C, C), 1)
    S2 = jnp.where(rowc == colc, 0.0, S2)

    near = jnp.zeros((C, C), jnp.float32)
    work = S2
    for _ in range(_NEAREST):
        m = jnp.min(work, axis=1, keepdims=True)
        eq = work == m
        first = jnp.min(jnp.where(eq, colc, C + 1), axis=1, keepdims=True)
        oh = colc == first
        near = jnp.where(oh, 1.0, near)
        work = jnp.where(oh, _INF, work)
    trust = 1.0 - near

    rowb = jax.lax.broadcasted_iota(jnp.int32, (B, C), 0)
    colb = jax.lax.broadcasted_iota(jnp.int32, (B, C), 1)
    lbl = rowb // _NUM
    onehot = (colb == lbl).astype(jnp.float32)
    tmask = jax.lax.dot_general(
        onehot, trust, (((1,), (1,)), ((), ())),
        preferred_element_type=jnp.float32)

    own = colb == lbl
    same = jnp.sum(jnp.where(own, D, 0.0), axis=1, keepdims=True)
    workd = jnp.where(own, _INF, D)

    found = jnp.zeros((B, 1), jnp.bool_)
    min_diff = jnp.zeros((B, 1), jnp.float32)
    for _ in range(_MAX_ITER):
        m = jnp.min(workd, axis=1, keepdims=True)
        eq = workd == m
        first = jnp.min(jnp.where(eq, colb, C + 1), axis=1, keepdims=True)
        oh = colb == first
        tr = jnp.sum(jnp.where(oh, tmask, 0.0), axis=1, keepdims=True) > 0.5
        take = jnp.logical_and(tr, jnp.logical_not(found))
        min_diff = jnp.where(take, m, min_diff)
        found = jnp.logical_or(found, tr)
        workd = jnp.where(oh, _INF, workd)

    hinge = jnp.maximum(_MARGIN + same - min_diff, 0.0)
    out_ref[0, 0] = jnp.sum(hinge) / B


def kernel(feature, centers):
    out = pl.pallas_call(
        _ipfl_body,
        out_shape=jax.ShapeDtypeStruct((1, 1), jnp.float32),
        out_specs=pl.BlockSpec(memory_space=pltpu.SMEM),
    )(feature, centers)
    return out[0, 0]


# speedup vs baseline: 1125.2514x; 1.8592x over previous
import jax
import jax.numpy as jnp
from jax.experimental import pallas as pl
from jax.experimental.pallas import tpu as pltpu

_MARGIN = 1.0
_MAX_ITER = 15
_NEAREST = 3
_NUM = 2
_INF = float("inf")
_BIG = 1e30


def _ipfl_body(f_ref, c_ref, out_ref):
    f = f_ref[:]
    c = c_ref[:]
    B, K = f.shape
    C = c.shape[0]

    hi = jax.lax.Precision.HIGHEST
    fn = jnp.sum(f * f, axis=1, keepdims=True)
    ones_row = jnp.ones((1, K), jnp.float32)
    cn_row = jax.lax.dot_general(
        ones_row, c * c, (((1,), (1,)), ((), ())),
        precision=hi, preferred_element_type=jnp.float32)
    fc = jax.lax.dot_general(
        f, c, (((1,), (1,)), ((), ())),
        precision=hi, preferred_element_type=jnp.float32)
    D2 = jnp.maximum(fn + cn_row - 2.0 * fc, 0.0)
    D = jnp.sqrt(D2)

    cc = jax.lax.dot_general(
        c, c, (((1,), (1,)), ((), ())),
        precision=hi, preferred_element_type=jnp.float32)
    cn_col = jax.lax.dot_general(
        c * c, ones_row, (((1,), (1,)), ((), ())),
        precision=hi, preferred_element_type=jnp.float32)
    S2 = jnp.maximum(cn_col + cn_row - 2.0 * cc, 0.0)
    rowc = jax.lax.broadcasted_iota(jnp.int32, (C, C), 0)
    colc = jax.lax.broadcasted_iota(jnp.int32, (C, C), 1)
    S2 = jnp.where(rowc == colc, 0.0, S2)

    near = jnp.zeros((C, C), jnp.float32)
    work = S2
    for _ in range(_NEAREST):
        m = jnp.min(work, axis=1, keepdims=True)
        eq = work == m
        first = jnp.min(jnp.where(eq, colc, C + 1), axis=1, keepdims=True)
        oh = colc == first
        near = jnp.where(oh, 1.0, near)
        work = jnp.where(oh, _INF, work)
    trust = 1.0 - near

    rowb = jax.lax.broadcasted_iota(jnp.int32, (B, C), 0)
    colb = jax.lax.broadcasted_iota(jnp.int32, (B, C), 1)
    lbl = rowb // _NUM
    onehot = (colb == lbl).astype(jnp.float32)
    tmask = jax.lax.dot_general(
        onehot, trust, (((1,), (1,)), ((), ())),
        preferred_element_type=jnp.float32)

    own = colb == lbl
    same = jnp.sum(jnp.where(own, D, 0.0), axis=1, keepdims=True)
    workd = jnp.where(own, _BIG, D)

    u = jnp.min(jnp.where(tmask > 0.5, workd, _BIG), axis=1, keepdims=True)
    cnt = jnp.sum((workd < u).astype(jnp.float32), axis=1, keepdims=True)
    found = jnp.logical_and(cnt < _MAX_ITER, u < _BIG * 0.5)
    min_diff = jnp.where(found, u, 0.0)

    hinge = jnp.maximum(_MARGIN + same - min_diff, 0.0)
    out_ref[0, 0] = jnp.sum(hinge) / B


def kernel(feature, centers):
    out = pl.pallas_call(
        _ipfl_body,
        out_shape=jax.ShapeDtypeStruct((1, 1), jnp.float32),
        out_specs=pl.BlockSpec(memory_space=pltpu.SMEM),
    )(feature, centers)
    return out[0, 0]
